# Initial kernel scaffold; baseline (speedup 1.0000x reference)
import jax
import jax.numpy as jnp
from jax.experimental import pallas as pl


def kernel(x, Wc, bc, W1, b1, W2, b2):
    raise NotImplementedError("write your pallas kernel here")


# speedup vs baseline: 2.0921x; 2.0921x over previous
import functools

import jax
import jax.numpy as jnp
from jax import lax
from jax.experimental import pallas as pl
from jax.experimental.pallas import tpu as pltpu

B, T, H, W = 4, 64, 224, 224
C = 4
CIN = 3
HW = H * W
HH, WW = 392, 128
NSEL = 4
TB = 4


def _score_sums_kernel(wc_ref, bc_ref, x_ref, y_ref):
    t = pl.program_id(1)
    xb = x_ref[0]
    x0, x1, x2 = xb[0], xb[1], xb[2]
    tot = jnp.zeros((TB, HH, WW), dtype=jnp.float32)
    for o in range(C):
        e = x0 * wc_ref[o, 0] + x1 * wc_ref[o, 1] + x2 * wc_ref[o, 2] + bc_ref[o]
        tot = tot + jnp.maximum(e, 0.0)
    lane = lax.broadcasted_iota(jnp.int32, (1, T), 1)
    row = jnp.zeros((1, T), dtype=jnp.float32)
    for i in range(TB):
        si = jnp.sum(tot[i])
        row = row + jnp.where(lane == t * TB + i, si, 0.0)

    @pl.when(t == 0)
    def _():
        y_ref[...] = jnp.zeros_like(y_ref)

    y_ref[...] = y_ref[...] + row[None]


def _select_kernel(y_ref, w1_ref, b1_ref, w2_ref, b2_ref, idx_ref):
    y = y_ref[...] * (1.0 / (C * HW))
    h = jnp.maximum(
        lax.dot_general(y, w1_ref[...], (((1,), (1,)), ((), ())),
                        preferred_element_type=jnp.float32) + b1_ref[...], 0.0)
    s = jax.nn.sigmoid(
        lax.dot_general(h, w2_ref[...], (((1,), (1,)), ((), ())),
                        preferred_element_type=jnp.float32) + b2_ref[...])
    iota = lax.broadcasted_iota(jnp.int32, (B, T), 1)
    picks = []
    cur = s
    for _ in range(NSEL):
        m = jnp.max(cur, axis=1, keepdims=True)
        idxv = jnp.min(jnp.where(cur == m, iota, T), axis=1)
        picks.append(idxv)
        cur = jnp.where(iota == idxv[:, None], -1.0, cur)
    cur = s
    for _ in range(NSEL):
        m = jnp.min(cur, axis=1, keepdims=True)
        idxv = jnp.min(jnp.where(cur == m, iota, T), axis=1)
        picks.append(idxv)
        cur = jnp.where(iota == idxv[:, None], 2.0, cur)
    idx_ref[...] = jnp.stack(picks, axis=1)


def _gather_embed_kernel(idx_ref, wc_ref, bc_ref, x_ref, out_ref):
    del idx_ref
    xb = x_ref[0, :, 0]
    x0, x1, x2 = xb[0], xb[1], xb[2]
    for o in range(C):
        e = x0 * wc_ref[o, 0] + x1 * wc_ref[o, 1] + x2 * wc_ref[o, 2] + bc_ref[o]
        out_ref[0, o, 0] = jnp.maximum(e, 0.0)


def kernel(x, Wc, bc, W1, b1, W2, b2):
    xf = x.reshape(B, CIN, T, HH, WW)

    y_sums = pl.pallas_call(
        _score_sums_kernel,
        grid=(B, T // TB),
        in_specs=[
            pl.BlockSpec(memory_space=pltpu.SMEM),
            pl.BlockSpec(memory_space=pltpu.SMEM),
            pl.BlockSpec((1, CIN, TB, HH, WW), lambda b, t: (b, 0, t, 0, 0)),
        ],
        out_specs=pl.BlockSpec((1, 1, T), lambda b, t: (b, 0, 0)),
        out_shape=jax.ShapeDtypeStruct((B, 1, T), jnp.float32),
    )(Wc, bc, xf)

    idx = pl.pallas_call(
        _select_kernel,
        out_shape=jax.ShapeDtypeStruct((B, 2 * NSEL), jnp.int32),
    )(y_sums.reshape(B, T), W1, b1.reshape(1, 32), W2, b2.reshape(1, 64))

    out = pl.pallas_call(
        _gather_embed_kernel,
        grid_spec=pltpu.PrefetchScalarGridSpec(
            num_scalar_prefetch=1,
            grid=(B, 2 * NSEL),
            in_specs=[
                pl.BlockSpec(memory_space=pltpu.SMEM),
                pl.BlockSpec(memory_space=pltpu.SMEM),
                pl.BlockSpec((1, CIN, 1, HH, WW),
                             lambda b, j, idx_ref: (b, 0, idx_ref[b, j], 0, 0)),
            ],
            out_specs=pl.BlockSpec((1, C, 1, HH, WW),
                                   lambda b, j, idx_ref: (b, 0, j, 0, 0)),
        ),
        out_shape=jax.ShapeDtypeStruct((B, C, 2 * NSEL, HH, WW), jnp.float32),
    )(idx, Wc, bc, xf)

    return out.reshape(B, C, 2 * NSEL, H, W)


# speedup vs baseline: 2.1652x; 1.0349x over previous
import functools

import jax
import jax.numpy as jnp
from jax import lax
from jax.experimental import pallas as pl
from jax.experimental.pallas import tpu as pltpu

B, T, H, W = 4, 64, 224, 224
C = 4
CIN = 3
HW = H * W
HH, WW = 392, 128
NSEL = 4
TB = 8


def _score_sums_kernel(wc_ref, bc_ref, x_ref, y_ref):
    t = pl.program_id(1)
    xb = x_ref[0]
    x0, x1, x2 = xb[0], xb[1], xb[2]
    chans = []
    for o in range(C):
        e = x2 * wc_ref[o, 2] + bc_ref[o]
        e = x1 * wc_ref[o, 1] + e
        e = x0 * wc_ref[o, 0] + e
        chans.append(jnp.maximum(e, 0.0))
    tot = (chans[0] + chans[1]) + (chans[2] + chans[3])
    lane = lax.broadcasted_iota(jnp.int32, (1, T), 1)
    row = jnp.zeros((1, T), dtype=jnp.float32)
    for i in range(TB):
        si = jnp.sum(tot[i])
        row = row + jnp.where(lane == t * TB + i, si, 0.0)

    @pl.when(t == 0)
    def _():
        y_ref[...] = jnp.zeros_like(y_ref)

    y_ref[...] = y_ref[...] + row[None]


def _select_kernel(y_ref, w1_ref, b1_ref, w2_ref, b2_ref, idx_ref):
    y = y_ref[...] * (1.0 / (C * HW))
    h = jnp.maximum(
        lax.dot_general(y, w1_ref[...], (((1,), (1,)), ((), ())),
                        preferred_element_type=jnp.float32) + b1_ref[...], 0.0)
    s = jax.nn.sigmoid(
        lax.dot_general(h, w2_ref[...], (((1,), (1,)), ((), ())),
                        preferred_element_type=jnp.float32) + b2_ref[...])
    iota = lax.broadcasted_iota(jnp.int32, (B, T), 1)
    picks = []
    cur = s
    for _ in range(NSEL):
        m = jnp.max(cur, axis=1, keepdims=True)
        idxv = jnp.min(jnp.where(cur == m, iota, T), axis=1)
        picks.append(idxv)
        cur = jnp.where(iota == idxv[:, None], -1.0, cur)
    cur = s
    for _ in range(NSEL):
        m = jnp.min(cur, axis=1, keepdims=True)
        idxv = jnp.min(jnp.where(cur == m, iota, T), axis=1)
        picks.append(idxv)
        cur = jnp.where(iota == idxv[:, None], 2.0, cur)
    idx_ref[...] = jnp.stack(picks, axis=1)


def _gather_embed_kernel(idx_ref, wc_ref, bc_ref, x_ref, out_ref):
    del idx_ref
    xb = x_ref[0, :, 0]
    x0, x1, x2 = xb[0], xb[1], xb[2]
    for o in range(C):
        e = x0 * wc_ref[o, 0] + x1 * wc_ref[o, 1] + x2 * wc_ref[o, 2] + bc_ref[o]
        out_ref[0, o, 0] = jnp.maximum(e, 0.0)


def kernel(x, Wc, bc, W1, b1, W2, b2):
    xf = x.reshape(B, CIN, T, HH, WW)

    y_sums = pl.pallas_call(
        _score_sums_kernel,
        grid=(B, T // TB),
        in_specs=[
            pl.BlockSpec(memory_space=pltpu.SMEM),
            pl.BlockSpec(memory_space=pltpu.SMEM),
            pl.BlockSpec((1, CIN, TB, HH, WW), lambda b, t: (b, 0, t, 0, 0)),
        ],
        out_specs=pl.BlockSpec((1, 1, T), lambda b, t: (b, 0, 0)),
        out_shape=jax.ShapeDtypeStruct((B, 1, T), jnp.float32),
    )(Wc, bc, xf)

    idx = pl.pallas_call(
        _select_kernel,
        out_shape=jax.ShapeDtypeStruct((B, 2 * NSEL), jnp.int32),
    )(y_sums.reshape(B, T), W1, b1.reshape(1, 32), W2, b2.reshape(1, 64))

    out = pl.pallas_call(
        _gather_embed_kernel,
        grid_spec=pltpu.PrefetchScalarGridSpec(
            num_scalar_prefetch=1,
            grid=(B, 2 * NSEL),
            in_specs=[
                pl.BlockSpec(memory_space=pltpu.SMEM),
                pl.BlockSpec(memory_space=pltpu.SMEM),
                pl.BlockSpec((1, CIN, 1, HH, WW),
                             lambda b, j, idx_ref: (b, 0, idx_ref[b, j], 0, 0)),
            ],
            out_specs=pl.BlockSpec((1, C, 1, HH, WW),
                                   lambda b, j, idx_ref: (b, 0, j, 0, 0)),
        ),
        out_shape=jax.ShapeDtypeStruct((B, C, 2 * NSEL, HH, WW), jnp.float32),
    )(idx, Wc, bc, xf)

    return out.reshape(B, C, 2 * NSEL, H, W)


# speedup vs baseline: 5.6333x; 2.6018x over previous
import functools

import jax
import jax.numpy as jnp
from jax import lax
from jax.experimental import pallas as pl
from jax.experimental.pallas import tpu as pltpu

B, T, H, W = 4, 64, 224, 224
C = 4
CIN = 3
NSEL = 4
TB = 8


def _score_sums_kernel(wc_ref, bc_ref, x_ref, y_ref):
    t = pl.program_id(1)
    xb = x_ref[0]
    x0, x1, x2 = xb[0], xb[1], xb[2]
    chans = []
    for o in range(C):
        e = x2 * wc_ref[o, 2] + bc_ref[o]
        e = x1 * wc_ref[o, 1] + e
        e = x0 * wc_ref[o, 0] + e
        chans.append(jnp.maximum(e, 0.0))
    tot = (chans[0] + chans[1]) + (chans[2] + chans[3])
    lane = lax.broadcasted_iota(jnp.int32, (1, T), 1)
    row = jnp.zeros((1, T), dtype=jnp.float32)
    for i in range(TB):
        si = jnp.sum(tot[i])
        row = row + jnp.where(lane == t * TB + i, si, 0.0)

    @pl.when(t == 0)
    def _():
        y_ref[...] = jnp.zeros_like(y_ref)

    y_ref[...] = y_ref[...] + row[None]


def _select_kernel(y_ref, w1_ref, b1_ref, w2_ref, b2_ref, idx_ref):
    y = y_ref[...] * (1.0 / (C * H * W))
    h = jnp.maximum(
        lax.dot_general(y, w1_ref[...], (((1,), (1,)), ((), ())),
                        preferred_element_type=jnp.float32) + b1_ref[...], 0.0)
    s = jax.nn.sigmoid(
        lax.dot_general(h, w2_ref[...], (((1,), (1,)), ((), ())),
                        preferred_element_type=jnp.float32) + b2_ref[...])
    iota = lax.broadcasted_iota(jnp.int32, (B, T), 1)
    picks = []
    cur = s
    for _ in range(NSEL):
        m = jnp.max(cur, axis=1, keepdims=True)
        idxv = jnp.min(jnp.where(cur == m, iota, T), axis=1)
        picks.append(idxv)
        cur = jnp.where(iota == idxv[:, None], -1.0, cur)
    cur = s
    for _ in range(NSEL):
        m = jnp.min(cur, axis=1, keepdims=True)
        idxv = jnp.min(jnp.where(cur == m, iota, T), axis=1)
        picks.append(idxv)
        cur = jnp.where(iota == idxv[:, None], 2.0, cur)
    idx_ref[...] = jnp.stack(picks, axis=1)


def _gather_embed_kernel(idx_ref, wc_ref, bc_ref, x_ref, out_ref):
    del idx_ref
    xb = x_ref[0, :, 0]
    x0, x1, x2 = xb[0], xb[1], xb[2]
    for o in range(C):
        e = x2 * wc_ref[o, 2] + bc_ref[o]
        e = x1 * wc_ref[o, 1] + e
        e = x0 * wc_ref[o, 0] + e
        out_ref[0, o, 0] = jnp.maximum(e, 0.0)


def kernel(x, Wc, bc, W1, b1, W2, b2):
    y_sums = pl.pallas_call(
        _score_sums_kernel,
        grid=(B, T // TB),
        in_specs=[
            pl.BlockSpec(memory_space=pltpu.SMEM),
            pl.BlockSpec(memory_space=pltpu.SMEM),
            pl.BlockSpec((1, CIN, TB, H, W), lambda b, t: (b, 0, t, 0, 0)),
        ],
        out_specs=pl.BlockSpec((1, 1, T), lambda b, t: (b, 0, 0)),
        out_shape=jax.ShapeDtypeStruct((B, 1, T), jnp.float32),
    )(Wc, bc, x)

    idx = pl.pallas_call(
        _select_kernel,
        out_shape=jax.ShapeDtypeStruct((B, 2 * NSEL), jnp.int32),
    )(y_sums.reshape(B, T), W1, b1.reshape(1, 32), W2, b2.reshape(1, 64))

    out = pl.pallas_call(
        _gather_embed_kernel,
        grid_spec=pltpu.PrefetchScalarGridSpec(
            num_scalar_prefetch=1,
            grid=(B, 2 * NSEL),
            in_specs=[
                pl.BlockSpec(memory_space=pltpu.SMEM),
                pl.BlockSpec(memory_space=pltpu.SMEM),
                pl.BlockSpec((1, CIN, 1, H, W),
                             lambda b, j, idx_ref: (b, 0, idx_ref[b, j], 0, 0)),
            ],
            out_specs=pl.BlockSpec((1, C, 1, H, W),
                                   lambda b, j, idx_ref: (b, 0, j, 0, 0)),
        ),
        out_shape=jax.ShapeDtypeStruct((B, C, 2 * NSEL, H, W), jnp.float32),
    )(idx, Wc, bc, x)

    return out
